# baseline (speedup 1.0000x reference)
import functools

import jax
import jax.numpy as jnp
from jax import lax
from jax.experimental import pallas as pl
from jax.experimental.pallas import tpu as pltpu
from jax.experimental.pallas import tpu_sc as plsc

N_NODES = 10000
N_EDGES = 320000
D = 128
N_ETYPES = 8

NC = 2
NS = 16
NW = NC * NS
E_PER_W = N_EDGES // NW
B = 80
NB = E_PER_W // B
LANES = 16
CHUNKS = D // LANES


def _rot(x, idx):
    return lax.gather(
        x, idx[:, None],
        dimension_numbers=lax.GatherDimensionNumbers(
            offset_dims=(), collapsed_slice_dims=(0,), start_index_map=(0,)),
        slice_sizes=(1,),
        mode=lax.GatherScatterMode.PROMISE_IN_BOUNDS)


def _distmul_body(h_hbm, u_hbm, v_hbm, t_hbm, w_hbm, out_hbm,
                  u_v, v_v, t_v, ru0_v, rv0_v, rw0_v, ru1_v, rv1_v, rw1_v,
                  sc_v, sem0, sem1):
    wid = lax.axis_index("s") * NC + lax.axis_index("c")
    base = wid * E_PER_W

    pltpu.sync_copy(u_hbm.at[pl.ds(base, E_PER_W)], u_v)
    pltpu.sync_copy(v_hbm.at[pl.ds(base, E_PER_W)], v_v)
    pltpu.sync_copy(t_hbm.at[pl.ds(base, E_PER_W)], t_v)

    iota = lax.iota(jnp.int32, LANES)
    rots = [jnp.bitwise_and(iota + k, LANES - 1) for k in (8, 4, 2, 1)]
    bufs = ((ru0_v, rv0_v, rw0_v, sem0), (ru1_v, rv1_v, rw1_v, sem1))

    def copies(g, ru, rv, rw, sem):
        e0 = g * B
        return (
            pltpu.make_async_copy(h_hbm.at[u_v.at[pl.ds(e0, B)]], ru, sem),
            pltpu.make_async_copy(h_hbm.at[v_v.at[pl.ds(e0, B)]], rv, sem),
            pltpu.make_async_copy(w_hbm.at[t_v.at[pl.ds(e0, B)]], rw, sem),
        )

    def start_block(g, *buf):
        for c in copies(g, *buf):
            c.start()

    def wait_block(g, *buf):
        for c in copies(g, *buf):
            c.wait()

    def compute_block(g, ru_v, rv_v, rw_v, sem):
        e0 = g * B

        def group_body(gg, gcarry):
            eg = gg * LANES

            def edge_body(ej, s):
                e = eg + ej
                acc = (ru_v[e, pl.ds(0, LANES)]
                       * rv_v[e, pl.ds(0, LANES)]
                       * rw_v[e, pl.ds(0, LANES)])
                for k in range(1, CHUNKS):
                    acc = acc + (ru_v[e, pl.ds(k * LANES, LANES)]
                                 * rv_v[e, pl.ds(k * LANES, LANES)]
                                 * rw_v[e, pl.ds(k * LANES, LANES)])
                r = acc
                for rv in rots:
                    r = r + _rot(r, rv)
                return jnp.where(iota == ej, r, s)

            s = lax.fori_loop(0, LANES, edge_body,
                              jnp.zeros((LANES,), jnp.float32), unroll=2)
            sig = 1.0 / (1.0 + jnp.exp(-s))
            sc_v[pl.ds(e0 + eg, LANES)] = sig
            return gcarry

        lax.fori_loop(0, B // LANES, group_body, 0)

    start_block(0, *bufs[0])
    start_block(1, *bufs[1])

    def pair_body(i, carry):
        g0 = 2 * i
        wait_block(g0, *bufs[0])
        compute_block(g0, *bufs[0])
        start_block(g0 + 2, *bufs[0])
        g1 = g0 + 1
        wait_block(g1, *bufs[1])
        compute_block(g1, *bufs[1])

        @pl.when(g1 + 2 < NB)
        def _():
            start_block(g1 + 2, *bufs[1])

        return carry

    lax.fori_loop(0, (NB - 1) // 2, pair_body, 0)
    wait_block(NB - 1, *bufs[0])
    compute_block(NB - 1, *bufs[0])

    pltpu.sync_copy(sc_v, out_hbm.at[pl.ds(base, E_PER_W)])


_distmul = functools.partial(
    pl.kernel,
    mesh=plsc.VectorSubcoreMesh(core_axis_name="c", subcore_axis_name="s"),
    out_type=jax.ShapeDtypeStruct((N_EDGES,), jnp.float32),
    scratch_types=[
        pltpu.VMEM((E_PER_W,), jnp.int32),
        pltpu.VMEM((E_PER_W,), jnp.int32),
        pltpu.VMEM((E_PER_W,), jnp.int32),
        pltpu.VMEM((B, D), jnp.float32),
        pltpu.VMEM((B, D), jnp.float32),
        pltpu.VMEM((B, D), jnp.float32),
        pltpu.VMEM((B, D), jnp.float32),
        pltpu.VMEM((B, D), jnp.float32),
        pltpu.VMEM((B, D), jnp.float32),
        pltpu.VMEM((E_PER_W,), jnp.float32),
        pltpu.SemaphoreType.DMA,
        pltpu.SemaphoreType.DMA,
    ],
)(_distmul_body)


def kernel(h, u, v, etype, rel_weight):
    return _distmul(h, u.astype(jnp.int32), v.astype(jnp.int32),
                    etype.astype(jnp.int32), rel_weight)


# speedup vs baseline: 4.1184x; 4.1184x over previous
import functools

import jax
import jax.numpy as jnp
from jax import lax
from jax.experimental import pallas as pl
from jax.experimental.pallas import tpu as pltpu
from jax.experimental.pallas import tpu_sc as plsc

N_NODES = 10000
N_EDGES = 320000
D = 128
N_ETYPES = 8

NC = 2
NS = 16
NW = NC * NS
E_PER_W = N_EDGES // NW
B = 80
NB = E_PER_W // B
LANES = 16
CHUNKS = D // LANES

PACK_ROWS = 200
V_BASE = N_ETYPES * N_NODES


def _pack_body(h_ref, w_ref, o_ref):
    o_ref[...] = h_ref[...] * w_ref[pl.ds(pl.program_id(0), 1), :]


_pack = pl.pallas_call(
    _pack_body,
    grid=(N_ETYPES + 1, N_NODES // PACK_ROWS),
    in_specs=[
        pl.BlockSpec((PACK_ROWS, D), lambda t, r: (r, 0)),
        pl.BlockSpec((N_ETYPES + 1, D), lambda t, r: (0, 0)),
    ],
    out_specs=pl.BlockSpec(
        (PACK_ROWS, D),
        lambda t, r: (t * (N_NODES // PACK_ROWS) + r, 0)),
    out_shape=jax.ShapeDtypeStruct(((N_ETYPES + 1) * N_NODES, D), jnp.float32),
)


def _rot(x, idx):
    return lax.gather(
        x, idx[:, None],
        dimension_numbers=lax.GatherDimensionNumbers(
            offset_dims=(), collapsed_slice_dims=(0,), start_index_map=(0,)),
        slice_sizes=(1,),
        mode=lax.GatherScatterMode.PROMISE_IN_BOUNDS)


def _distmul_body(tab_hbm, u_hbm, v_hbm, t_hbm, out_hbm,
                  u_v, v_v, t_v, ru0_v, rv0_v, ru1_v, rv1_v,
                  sc_v, sem0, sem1):
    wid = lax.axis_index("s") * NC + lax.axis_index("c")
    base = wid * E_PER_W

    pltpu.sync_copy(u_hbm.at[pl.ds(base, E_PER_W)], u_v)
    pltpu.sync_copy(v_hbm.at[pl.ds(base, E_PER_W)], v_v)
    pltpu.sync_copy(t_hbm.at[pl.ds(base, E_PER_W)], t_v)

    def xform_body(i, carry):
        sl = pl.ds(i * LANES, LANES)
        u_v[sl] = u_v[sl] + t_v[sl] * N_NODES
        v_v[sl] = v_v[sl] + V_BASE
        return carry

    lax.fori_loop(0, E_PER_W // LANES, xform_body, 0, unroll=4)

    iota = lax.iota(jnp.int32, LANES)
    rots = [jnp.bitwise_and(iota + k, LANES - 1) for k in (8, 4, 2, 1)]
    bufs = ((ru0_v, rv0_v, sem0), (ru1_v, rv1_v, sem1))

    def copies(g, ru, rv, sem):
        e0 = g * B
        return (
            pltpu.make_async_copy(tab_hbm.at[u_v.at[pl.ds(e0, B)]], ru, sem),
            pltpu.make_async_copy(tab_hbm.at[v_v.at[pl.ds(e0, B)]], rv, sem),
        )

    def start_block(g, *buf):
        for c in copies(g, *buf):
            c.start()

    def wait_block(g, *buf):
        for c in copies(g, *buf):
            c.wait()

    def compute_block(g, ru_v, rv_v, sem):
        e0 = g * B

        def group_body(gg, gcarry):
            eg = gg * LANES

            def edge_body(ej, s):
                e = eg + ej
                acc = (ru_v[e, pl.ds(0, LANES)]
                       * rv_v[e, pl.ds(0, LANES)])
                for k in range(1, CHUNKS):
                    acc = acc + (ru_v[e, pl.ds(k * LANES, LANES)]
                                 * rv_v[e, pl.ds(k * LANES, LANES)])
                r = acc
                for rv in rots:
                    r = r + _rot(r, rv)
                return jnp.where(iota == ej, r, s)

            s = lax.fori_loop(0, LANES, edge_body,
                              jnp.zeros((LANES,), jnp.float32), unroll=2)
            sig = 1.0 / (1.0 + jnp.exp(-s))
            sc_v[pl.ds(e0 + eg, LANES)] = sig
            return gcarry

        lax.fori_loop(0, B // LANES, group_body, 0)

    start_block(0, *bufs[0])
    start_block(1, *bufs[1])

    def pair_body(i, carry):
        g0 = 2 * i
        wait_block(g0, *bufs[0])
        compute_block(g0, *bufs[0])
        start_block(g0 + 2, *bufs[0])
        g1 = g0 + 1
        wait_block(g1, *bufs[1])
        compute_block(g1, *bufs[1])

        @pl.when(g1 + 2 < NB)
        def _():
            start_block(g1 + 2, *bufs[1])

        return carry

    lax.fori_loop(0, (NB - 1) // 2, pair_body, 0)
    wait_block(NB - 1, *bufs[0])
    compute_block(NB - 1, *bufs[0])

    pltpu.sync_copy(sc_v, out_hbm.at[pl.ds(base, E_PER_W)])


_distmul = functools.partial(
    pl.kernel,
    mesh=plsc.VectorSubcoreMesh(core_axis_name="c", subcore_axis_name="s"),
    out_type=jax.ShapeDtypeStruct((N_EDGES,), jnp.float32),
    scratch_types=[
        pltpu.VMEM((E_PER_W,), jnp.int32),
        pltpu.VMEM((E_PER_W,), jnp.int32),
        pltpu.VMEM((E_PER_W,), jnp.int32),
        pltpu.VMEM((B, D), jnp.float32),
        pltpu.VMEM((B, D), jnp.float32),
        pltpu.VMEM((B, D), jnp.float32),
        pltpu.VMEM((B, D), jnp.float32),
        pltpu.VMEM((E_PER_W,), jnp.float32),
        pltpu.SemaphoreType.DMA,
        pltpu.SemaphoreType.DMA,
    ],
)(_distmul_body)


def kernel(h, u, v, etype, rel_weight):
    w9 = jnp.concatenate(
        [rel_weight, jnp.ones((1, D), jnp.float32)], axis=0)
    tab = _pack(h, w9)
    return _distmul(tab, u.astype(jnp.int32), v.astype(jnp.int32),
                    etype.astype(jnp.int32))
